# baseline (speedup 1.0000x reference)
import functools

import jax
import jax.numpy as jnp
from jax import lax
from jax.experimental import pallas as pl
from jax.experimental.pallas import tpu as pltpu
from jax.experimental.pallas import tpu_sc as plsc

BATCH = 16384
EMB = 128
LANES = 16
GROUPS = EMB // LANES
CHUNK = 128


def kernel(embedded_x, c, masks):
    info = plsc.get_sparse_core_info()
    n_workers = info.num_cores * info.num_subcores
    b_per_w = BATCH // n_workers
    n_chunks = b_per_w // CHUNK

    mesh = plsc.VectorSubcoreMesh(core_axis_name="c", subcore_axis_name="s")

    @functools.partial(
        pl.kernel,
        mesh=mesh,
        out_type=jax.ShapeDtypeStruct((BATCH, EMB), jnp.float32),
        scratch_types=[
            pltpu.VMEM((b_per_w,), jnp.int32),
            pltpu.VMEM((CHUNK, EMB), jnp.float32),
            pltpu.VMEM((CHUNK, EMB), jnp.float32),
            pltpu.SemaphoreType.DMA,
        ],
    )
    def run(x_hbm, c_hbm, m_hbm, out_hbm, idx_v, x_v, m_v, sem):
        wid = lax.axis_index("s") * info.num_cores + lax.axis_index("c")
        base = wid * b_per_w
        pltpu.sync_copy(c_hbm.at[pl.ds(base, b_per_w)], idx_v)
        for j in range(n_chunks):
            rbase = base + j * CHUNK
            gather = pltpu.async_copy(
                m_hbm.at[idx_v.at[pl.ds(j * CHUNK, CHUNK)]], m_v, sem)
            pltpu.sync_copy(x_hbm.at[pl.ds(rbase, CHUNK)], x_v)
            gather.wait()

            def row_body(r, carry):
                for g in range(GROUPS):
                    sl = pl.ds(g * LANES, LANES)
                    x_v[r, sl] = x_v[r, sl] * m_v[r, sl]
                return carry

            lax.fori_loop(0, CHUNK, row_body, 0)
            pltpu.sync_copy(x_v, out_hbm.at[pl.ds(rbase, CHUNK)])

    return run(embedded_x, c.astype(jnp.int32), masks)


# speedup vs baseline: 2.1368x; 2.1368x over previous
import functools

import jax
import jax.numpy as jnp
from jax import lax
from jax.experimental import pallas as pl
from jax.experimental.pallas import tpu as pltpu
from jax.experimental.pallas import tpu_sc as plsc

BATCH = 16384
EMB = 128
LANES = 16
GROUPS = EMB // LANES
CHUNK = 128
N_COND = 8


def kernel(embedded_x, c, masks):
    info = plsc.get_sparse_core_info()
    n_workers = info.num_cores * info.num_subcores
    b_per_w = BATCH // n_workers
    n_chunks = b_per_w // CHUNK

    mesh = plsc.VectorSubcoreMesh(core_axis_name="c", subcore_axis_name="s")

    @functools.partial(
        pl.kernel,
        mesh=mesh,
        out_type=jax.ShapeDtypeStruct((BATCH, EMB), jnp.float32),
        scratch_types=[
            pltpu.VMEM((b_per_w,), jnp.int32),
            pltpu.VMEM((N_COND, EMB), jnp.float32),
        ]
        + [pltpu.VMEM((CHUNK, EMB), jnp.float32) for _ in range(n_chunks)]
        + [pltpu.SemaphoreType.DMA for _ in range(2 * n_chunks)],
    )
    def run(x_hbm, c_hbm, m_hbm, out_hbm, idx_v, masks_v, *rest):
        bufs = rest[:n_chunks]
        load_sems = rest[n_chunks:2 * n_chunks]
        store_sems = rest[2 * n_chunks:]

        wid = lax.axis_index("s") * info.num_cores + lax.axis_index("c")
        base = wid * b_per_w
        pltpu.sync_copy(c_hbm.at[pl.ds(base, b_per_w)], idx_v)
        pltpu.sync_copy(m_hbm, masks_v)

        loads = [
            pltpu.async_copy(
                x_hbm.at[pl.ds(base + j * CHUNK, CHUNK)], bufs[j], load_sems[j])
            for j in range(n_chunks)
        ]

        stores = []
        for j in range(n_chunks):
            loads[j].wait()
            buf = bufs[j]

            def grp_body(t, carry, _j=j, _buf=buf):
                cvec = idx_v[pl.ds(_j * CHUNK + t * LANES, LANES)]
                for l in range(LANES):
                    rowc = cvec[l]
                    r = t * LANES + l
                    for g in range(GROUPS):
                        sl = pl.ds(g * LANES, LANES)
                        _buf[r, sl] = _buf[r, sl] * masks_v[rowc, sl]
                return carry

            lax.fori_loop(0, CHUNK // LANES, grp_body, 0)
            stores.append(
                pltpu.async_copy(
                    buf, out_hbm.at[pl.ds(base + j * CHUNK, CHUNK)],
                    store_sems[j]))
        for s in stores:
            s.wait()

    return run(embedded_x, c.astype(jnp.int32), masks)


# speedup vs baseline: 2.4243x; 1.1345x over previous
import functools

import jax
import jax.numpy as jnp
from jax import lax
from jax.experimental import pallas as pl
from jax.experimental.pallas import tpu as pltpu
from jax.experimental.pallas import tpu_sc as plsc

BATCH = 16384
EMB = 128
LANES = 16
GROUPS = EMB // LANES
CHUNK = 128
N_COND = 8
N_OBUF = 2


def kernel(embedded_x, c, masks):
    info = plsc.get_sparse_core_info()
    n_workers = info.num_cores * info.num_subcores
    b_per_w = BATCH // n_workers
    n_chunks = b_per_w // CHUNK

    mesh = plsc.VectorSubcoreMesh(core_axis_name="c", subcore_axis_name="s")

    @functools.partial(
        pl.kernel,
        mesh=mesh,
        out_type=jax.ShapeDtypeStruct((BATCH, EMB), jnp.float32),
        scratch_types=[
            pltpu.VMEM((b_per_w,), jnp.int32),
            pltpu.VMEM((N_COND, EMB), jnp.float32),
        ]
        + [pltpu.VMEM((CHUNK, EMB), jnp.float32) for _ in range(n_chunks)]
        + [pltpu.VMEM((CHUNK, EMB), jnp.float32) for _ in range(N_OBUF)]
        + [pltpu.SemaphoreType.DMA for _ in range(n_chunks + N_OBUF + 2)],
    )
    def run(x_hbm, c_hbm, m_hbm, out_hbm, idx_v, masks_v, *rest):
        xbufs = rest[:n_chunks]
        obufs = rest[n_chunks:n_chunks + N_OBUF]
        sems = rest[n_chunks + N_OBUF:]
        load_sems = sems[:n_chunks]
        store_sems = sems[n_chunks:n_chunks + N_OBUF]
        idx_sem, msk_sem = sems[n_chunks + N_OBUF:]

        wid = lax.axis_index("s") * info.num_cores + lax.axis_index("c")
        base = wid * b_per_w

        idx_cp = pltpu.async_copy(
            c_hbm.at[pl.ds(base, b_per_w)], idx_v, idx_sem)
        msk_cp = pltpu.async_copy(m_hbm, masks_v, msk_sem)
        loads = [
            pltpu.async_copy(
                x_hbm.at[pl.ds(base + j * CHUNK, CHUNK)], xbufs[j],
                load_sems[j])
            for j in range(n_chunks)
        ]
        idx_cp.wait()
        msk_cp.wait()

        stores = [None] * n_chunks
        for j in range(n_chunks):
            xb = xbufs[j]
            ob = obufs[j % N_OBUF]
            if j >= N_OBUF:
                stores[j - N_OBUF].wait()
            loads[j].wait()

            @plsc.parallel_loop(0, CHUNK // LANES, unroll=2)
            def grp_body(t, _j=j, _xb=xb, _ob=ob):
                cvec = idx_v[pl.ds(_j * CHUNK + t * LANES, LANES)]
                for l in range(LANES):
                    rowc = cvec[l]
                    r = t * LANES + l
                    for g in range(GROUPS):
                        sl = pl.ds(g * LANES, LANES)
                        _ob[r, sl] = _xb[r, sl] * masks_v[rowc, sl]

            stores[j] = pltpu.async_copy(
                ob, out_hbm.at[pl.ds(base + j * CHUNK, CHUNK)],
                store_sems[j % N_OBUF])
        for j in range(n_chunks - N_OBUF, n_chunks):
            stores[j].wait()

    return run(embedded_x, c.astype(jnp.int32), masks)
